# baseline (speedup 1.0000x reference)
import functools

import jax
import jax.numpy as jnp
from jax import lax
from jax.experimental import pallas as pl
from jax.experimental.pallas import tpu as pltpu
from jax.experimental.pallas import tpu_sc as plsc

_V = 3
_C2D = 64
_D3 = 96
_BS = 2
_H = 120
_W = 160
_N = 100000
_HW = _H * _W
_PAD_ROWS = 1920
_SLAB = _HW + _PAD_ROWS
_ZROW = _HW
_NVB = _V * _BS
_TROWS = _NVB * _SLAB

_NW = 32
_K = 256
_NBLK = 14
_NPAD = _NW * _K * _NBLK

_BN = 2000
_NTB = _N // _BN
_EPS = 1e-5


_TBLK = 1920


def _transpose_body(in_ref, out_ref):
    j = pl.program_id(1)

    @pl.when(j < 10)
    def _():
        out_ref[0] = in_ref[0].T

    @pl.when(j == 10)
    def _():
        out_ref[0] = jnp.zeros((_TBLK, _C2D), jnp.float32)


def _build_table(feat6):
    return pl.pallas_call(
        _transpose_body,
        grid=(_NVB, _SLAB // _TBLK),
        in_specs=[pl.BlockSpec((1, _C2D, _TBLK),
                               lambda i, j: (i, 0, jnp.minimum(j, 9)))],
        out_specs=pl.BlockSpec((1, _TBLK, _C2D), lambda i, j: (i, j, 0)),
        out_shape=jax.ShapeDtypeStruct((_NVB, _SLAB, _C2D), jnp.float32),
    )(feat6)


def _sc_body(table_hbm, rowids_hbm, maps_hbm,
             x0_hbm, x1_hbm, x2_hbm,
             m0, m1, m2, r0, r1, r2,
             i00, i01, i02, i10, i11, i12, i20, i21, i22,
             f00, f01, f02, f10, f11, f12,
             ms0, ms1, ms2, gs0, gs1, gs2, fs0, fs1, ss0, ss1):
    wid = lax.axis_index("s") * 2 + lax.axis_index("c")
    maps = (m0, m1, m2)
    rids = (r0, r1, r2)
    idxs = ((i00, i01, i02), (i10, i11, i12), (i20, i21, i22))
    feats = ((f00, f01, f02), (f10, f11, f12))
    msems = (ms0, ms1, ms2)
    gsems = (gs0, gs1, gs2)
    fsems = (fs0, fs1)
    ssems = (ss0, ss1)
    xs = (x0_hbm, x1_hbm, x2_hbm)

    def fire_map(b):
        d = b % 3
        gb = wid + _NW * b
        return pltpu.make_async_copy(
            maps_hbm.at[pl.ds(gb * 4, 4)], maps[d], msems[d])

    def fire_rid(b):
        d = b % 3
        return [pltpu.make_async_copy(
            rowids_hbm.at[maps[d].at[sub]],
            rids[d].at[pl.ds(sub * 128, 128)], gsems[d]) for sub in range(2)]

    def compute_idx(b):
        d = b % 3
        for sub in range(2):
            for g in range(8):
                rows = lax.iota(jnp.int32, 16) + (sub * 128 + g * 16)
                for v in range(_V):
                    cv = jnp.full((16,), v, jnp.int32)
                    idxs[d][v][sub, pl.ds(g * 16, 16)] = plsc.load_gather(
                        rids[d], [rows, cv])

    def fire_fgather(b):
        d, p = b % 3, b % 2
        return [pltpu.make_async_copy(
            table_hbm.at[idxs[d][v].at[sub]],
            feats[p][v].at[pl.ds(sub * 128, 128)], fsems[p])
            for v in range(_V) for sub in range(2)]

    def fire_scatter(b):
        d, p = b % 3, b % 2
        return [pltpu.make_async_copy(
            feats[p][v].at[pl.ds(sub * 128, 128)],
            xs[v].at[maps[d].at[2 + sub]], ssems[p])
            for v in range(_V) for sub in range(2)]

    def start(copies):
        for c in copies:
            c.start()
        return copies

    def drain(copies):
        for c in copies:
            c.wait()

    c = fire_map(0)
    c.start()
    c.wait()
    drain(start(fire_rid(0)))
    compute_idx(0)

    pending_s = {0: [], 1: []}
    for b in range(_NBLK):
        p = b % 2
        drain(pending_s[p])
        fg = start(fire_fgather(b))
        if b + 1 < _NBLK:
            mc = fire_map(b + 1)
            mc.start()
        drain(fg)
        pending_s[p] = start(fire_scatter(b))
        if b + 1 < _NBLK:
            mc.wait()
            drain(start(fire_rid(b + 1)))
            compute_idx(b + 1)
    drain(pending_s[0])
    drain(pending_s[1])


def _rowid_body(lk_ref, out_ref):
    lk = lk_ref[...]
    cols = []
    for v in range(_V):
        b = lk[:, v:v + 1]
        vi = lk[:, 3 + v:4 + v]
        ui = lk[:, 6 + v:7 + v]
        val = lk[:, 9 + v:10 + v]
        pix = jnp.where(val > 0, vi * _W + ui, _ZROW)
        cols.append((v * _BS + b) * _SLAB + pix)
    cols.append(jnp.zeros((_BN, 16 - _V), jnp.int32))
    out_ref[...] = jnp.concatenate(cols, axis=1)


def _build_rowids(links2d):
    return pl.pallas_call(
        _rowid_body,
        grid=(_NTB,),
        in_specs=[pl.BlockSpec((_BN, 4 * _V), lambda i: (i, 0))],
        out_specs=pl.BlockSpec((_BN, 16), lambda i: (i, 0)),
        out_shape=jax.ShapeDtypeStruct((_N, 16), jnp.int32),
    )(links2d)


def _sc_gather(table2d, rowids, maps):
    mesh = plsc.VectorSubcoreMesh(core_axis_name="c", subcore_axis_name="s",
                                  num_cores=2, num_subcores=16)
    xshape = jax.ShapeDtypeStruct((_NPAD, _C2D), jnp.float32)
    fn = pl.kernel(
        _sc_body,
        out_type=[xshape, xshape, xshape],
        mesh=mesh,
        compiler_params=pltpu.CompilerParams(needs_layout_passes=False,
                                             use_tc_tiling_on_sc=False),
        scratch_types=(
            [pltpu.VMEM((4, 128), jnp.int32)] * 3
            + [pltpu.VMEM((_K, 16), jnp.int32)] * 3
            + [pltpu.VMEM((2, 128), jnp.int32)] * 9
            + [pltpu.VMEM((_K, _C2D), jnp.float32)] * 6
            + [pltpu.SemaphoreType.DMA] * 10
        ),
    )
    return fn(table2d, rowids, maps)


def _scale_shift(stats_ref, g_ref, be_ref):
    inv_n = jnp.float32(1.0 / _N)
    mu = stats_ref[0:1, :] * inv_n
    var = stats_ref[1:2, :] * inv_n - mu * mu
    sc = g_ref[...] * lax.rsqrt(var + _EPS)
    sh = be_ref[...] - mu * sc
    return sc, sh


def _accum(i, y, acc_ref, stats_ref):
    @pl.when(i == 0)
    def _():
        acc_ref[...] = jnp.zeros_like(acc_ref)

    acc_ref[0:1, :] += jnp.sum(y, axis=0, keepdims=True)
    acc_ref[1:2, :] += jnp.sum(y * y, axis=0, keepdims=True)

    @pl.when(i == _NTB - 1)
    def _():
        stats_ref[...] = acc_ref[...]


def _p1_body(x0, x1, x2, w1, b1, y1_out, stats_out, acc):
    i = pl.program_id(0)
    y = (jnp.dot(x0[...], w1[0:64, :], preferred_element_type=jnp.float32)
         + jnp.dot(x1[...], w1[64:128, :], preferred_element_type=jnp.float32)
         + jnp.dot(x2[...], w1[128:192, :], preferred_element_type=jnp.float32)
         + b1[...])
    y1_out[...] = y
    _accum(i, y, acc, stats_out)


def _h1(y1_ref, s1, g1, be1):
    sc1, sh1 = _scale_shift(s1, g1, be1)
    return jnp.maximum(y1_ref[...] * sc1 + sh1, 0.0)


def _p2_body(y1, s1, g1, be1, w2, b2, stats_out, acc):
    i = pl.program_id(0)
    h = _h1(y1, s1, g1, be1)
    y2 = jnp.dot(h, w2[...], preferred_element_type=jnp.float32) + b2[...]
    _accum(i, y2, acc, stats_out)


def _y3(y1, f3d, s1, g1, be1, w2, b2, s2, g2, be2, w3, b3):
    h = _h1(y1, s1, g1, be1)
    y2 = jnp.dot(h, w2[...], preferred_element_type=jnp.float32) + b2[...]
    sc2, sh2 = _scale_shift(s2, g2, be2)
    h2 = jnp.maximum(y2 * sc2 + sh2, 0.0)
    return (jnp.dot(f3d[...], w3[0:_D3, :], preferred_element_type=jnp.float32)
            + jnp.dot(h2, w3[_D3:2 * _D3, :], preferred_element_type=jnp.float32)
            + b3[...])


def _p3_body(y1, f3d, s1, g1, be1, w2, b2, s2, g2, be2, w3, b3,
             stats_out, acc):
    i = pl.program_id(0)
    y3 = _y3(y1, f3d, s1, g1, be1, w2, b2, s2, g2, be2, w3, b3)
    _accum(i, y3, acc, stats_out)


def _p4_body(y1, f3d, s1, g1, be1, w2, b2, s2, g2, be2, w3, b3, s3, g3, be3,
             out):
    y3 = _y3(y1, f3d, s1, g1, be1, w2, b2, s2, g2, be2, w3, b3)
    sc3, sh3 = _scale_shift(s3, g3, be3)
    out[...] = jnp.maximum(y3 * sc3 + sh3, 0.0)


def _row_spec(c):
    return pl.BlockSpec((_BN, c), lambda i: (i, 0))


def _full_spec(r, c):
    return pl.BlockSpec((r, c), lambda i: (0, 0))


def _stats_spec(c):
    return pl.BlockSpec((2, c), lambda i: (0, 0))


def kernel(feat_2d_all, sparse_feat_3d_F, links, coords_map_in,
           coords_map_out, W1, b1, g1, be1, W2, b2, g2, be2, W3, b3, g3,
           be3):
    feat6 = feat_2d_all.reshape(_NVB, _C2D, _HW)
    table = _build_table(feat6).reshape(_TROWS, _C2D)

    links2d = links.reshape(_N, 4 * _V)
    pad = _NPAD - _N
    cin_p = jnp.concatenate(
        [coords_map_in, jnp.zeros((pad,), jnp.int32)]).reshape(-1, 2, 128)
    cout_p = jnp.concatenate(
        [coords_map_out,
         jnp.arange(_N, _NPAD, dtype=jnp.int32)]).reshape(-1, 2, 128)
    maps = jnp.concatenate([cin_p, cout_p], axis=1).reshape(-1, 128)

    rowids = _build_rowids(links2d)
    x0, x1, x2 = _sc_gather(table, rowids, maps)

    r = lambda a: a.reshape(1, -1)
    b1r, g1r, be1r = r(b1), r(g1), r(be1)
    b2r, g2r, be2r = r(b2), r(g2), r(be2)
    b3r, g3r, be3r = r(b3), r(g3), r(be3)

    y1, s1 = pl.pallas_call(
        _p1_body,
        grid=(_NTB,),
        in_specs=[_row_spec(_C2D), _row_spec(_C2D), _row_spec(_C2D),
                  _full_spec(_V * _C2D, _C2D), _full_spec(1, _C2D)],
        out_specs=[_row_spec(_C2D), _stats_spec(_C2D)],
        out_shape=[jax.ShapeDtypeStruct((_N, _C2D), jnp.float32),
                   jax.ShapeDtypeStruct((2, _C2D), jnp.float32)],
        scratch_shapes=[pltpu.VMEM((2, _C2D), jnp.float32)],
    )(x0, x1, x2, W1, b1r)

    s2 = pl.pallas_call(
        _p2_body,
        grid=(_NTB,),
        in_specs=[_row_spec(_C2D), _stats_spec(_C2D), _full_spec(1, _C2D),
                  _full_spec(1, _C2D), _full_spec(_C2D, _D3),
                  _full_spec(1, _D3)],
        out_specs=_stats_spec(_D3),
        out_shape=jax.ShapeDtypeStruct((2, _D3), jnp.float32),
        scratch_shapes=[pltpu.VMEM((2, _D3), jnp.float32)],
    )(y1, s1, g1r, be1r, W2, b2r)

    big_in = [_row_spec(_C2D), _row_spec(_D3), _stats_spec(_C2D),
              _full_spec(1, _C2D), _full_spec(1, _C2D),
              _full_spec(_C2D, _D3), _full_spec(1, _D3), _stats_spec(_D3),
              _full_spec(1, _D3), _full_spec(1, _D3),
              _full_spec(2 * _D3, _D3), _full_spec(1, _D3)]

    s3 = pl.pallas_call(
        _p3_body,
        grid=(_NTB,),
        in_specs=big_in,
        out_specs=_stats_spec(_D3),
        out_shape=jax.ShapeDtypeStruct((2, _D3), jnp.float32),
        scratch_shapes=[pltpu.VMEM((2, _D3), jnp.float32)],
    )(y1, sparse_feat_3d_F, s1, g1r, be1r, W2, b2r, s2, g2r, be2r, W3, b3r)

    out = pl.pallas_call(
        _p4_body,
        grid=(_NTB,),
        in_specs=big_in + [_stats_spec(_D3), _full_spec(1, _D3),
                           _full_spec(1, _D3)],
        out_specs=_row_spec(_D3),
        out_shape=jax.ShapeDtypeStruct((_N, _D3), jnp.float32),
    )(y1, sparse_feat_3d_F, s1, g1r, be1r, W2, b2r, s2, g2r, be2r, W3, b3r,
      s3, g3r, be3r)

    return out


# speedup vs baseline: 1.0133x; 1.0133x over previous
import functools

import jax
import jax.numpy as jnp
from jax import lax
from jax.experimental import pallas as pl
from jax.experimental.pallas import tpu as pltpu
from jax.experimental.pallas import tpu_sc as plsc

_V = 3
_C2D = 64
_D3 = 96
_BS = 2
_H = 120
_W = 160
_N = 100000
_HW = _H * _W
_PAD_ROWS = 1920
_SLAB = _HW + _PAD_ROWS
_ZROW = _HW
_NVB = _V * _BS
_TROWS = _NVB * _SLAB

_NW = 32
_K = 256
_NBLK = 14
_NPAD = _NW * _K * _NBLK

_BN = 2000
_NTB = _N // _BN
_EPS = 1e-5


_TBLK = 1920


def _transpose_body(in_ref, out_ref):
    j = pl.program_id(1)

    @pl.when(j < 10)
    def _():
        out_ref[0] = in_ref[0].T

    @pl.when(j == 10)
    def _():
        out_ref[0] = jnp.zeros((_TBLK, _C2D), jnp.float32)


def _build_table(feat6):
    return pl.pallas_call(
        _transpose_body,
        grid=(_NVB, _SLAB // _TBLK),
        in_specs=[pl.BlockSpec((1, _C2D, _TBLK),
                               lambda i, j: (i, 0, jnp.minimum(j, 9)))],
        out_specs=pl.BlockSpec((1, _TBLK, _C2D), lambda i, j: (i, j, 0)),
        out_shape=jax.ShapeDtypeStruct((_NVB, _SLAB, _C2D), jnp.float32),
    )(feat6)


def _sc_body(table_hbm, rowids_hbm, maps_hbm,
             x0_hbm, x1_hbm, x2_hbm,
             m0, m1, r0, r1,
             i00, i01, i02, i10, i11, i12,
             f00, f01, f02, f10, f11, f12,
             ms0, ms1, gs0, gs1, fs0, fs1, ss0, ss1):
    wid = lax.axis_index("s") * 2 + lax.axis_index("c")
    maps = (m0, m1)
    rids = (r0, r1)
    idxs = ((i00, i01, i02), (i10, i11, i12))
    feats = ((f00, f01, f02), (f10, f11, f12))
    msems = (ms0, ms1)
    gsems = (gs0, gs1)
    fsems = (fs0, fs1)
    ssems = (ss0, ss1)
    xs = (x0_hbm, x1_hbm, x2_hbm)

    def map_copy(b, p):
        gb = wid + _NW * b
        return pltpu.make_async_copy(
            maps_hbm.at[pl.ds(gb * 4, 4)], maps[p], msems[p])

    def rid_copies(p):
        return [pltpu.make_async_copy(
            rowids_hbm.at[maps[p].at[sub]],
            rids[p].at[pl.ds(sub * 128, 128)], gsems[p]) for sub in range(2)]

    def compute_idx(p):
        for sub in range(2):
            for g in range(8):
                rows = lax.iota(jnp.int32, 16) + (sub * 128 + g * 16)
                for v in range(_V):
                    cv = jnp.full((16,), v, jnp.int32)
                    idxs[p][v][sub, pl.ds(g * 16, 16)] = plsc.load_gather(
                        rids[p], [rows, cv])

    def fgather_copies(p):
        return [pltpu.make_async_copy(
            table_hbm.at[idxs[p][v].at[sub]],
            feats[p][v].at[pl.ds(sub * 128, 128)], fsems[p])
            for v in range(_V) for sub in range(2)]

    def scatter_copies(p):
        return [pltpu.make_async_copy(
            feats[p][v].at[pl.ds(sub * 128, 128)],
            xs[v].at[maps[p].at[2 + sub]], ssems[p])
            for v in range(_V) for sub in range(2)]

    def start(copies):
        for c in copies:
            c.start()

    def drain(copies):
        for c in copies:
            c.wait()

    def prefetch(b1, q):
        map_copy(b1, q).start()
        map_copy(b1, q).wait()
        start(rid_copies(q))
        drain(rid_copies(q))
        compute_idx(q)

    def body(b, p, first, prefetch_next):
        q = 1 - p
        start(fgather_copies(p))
        if not first:
            drain(scatter_copies(q))
        drain(fgather_copies(p))
        start(scatter_copies(p))
        if prefetch_next:
            prefetch(b + 1, q)

    prefetch(0, 0)
    body(0, 0, True, True)

    def chunk(c, carry):
        b = 1 + 2 * c
        body(b, 1, False, True)
        body(b + 1, 0, False, True)
        return carry

    lax.fori_loop(0, (_NBLK - 2) // 2, chunk, 0)

    body(_NBLK - 1, 1, False, False)
    drain(scatter_copies(1))


def _rowid_body(lk_ref, out_ref):
    lk = lk_ref[...]
    cols = []
    for v in range(_V):
        b = lk[:, v:v + 1]
        vi = lk[:, 3 + v:4 + v]
        ui = lk[:, 6 + v:7 + v]
        val = lk[:, 9 + v:10 + v]
        pix = jnp.where(val > 0, vi * _W + ui, _ZROW)
        cols.append((v * _BS + b) * _SLAB + pix)
    cols.append(jnp.zeros((_BN, 16 - _V), jnp.int32))
    out_ref[...] = jnp.concatenate(cols, axis=1)


def _build_rowids(links2d):
    return pl.pallas_call(
        _rowid_body,
        grid=(_NTB,),
        in_specs=[pl.BlockSpec((_BN, 4 * _V), lambda i: (i, 0))],
        out_specs=pl.BlockSpec((_BN, 16), lambda i: (i, 0)),
        out_shape=jax.ShapeDtypeStruct((_N, 16), jnp.int32),
    )(links2d)


def _sc_gather(table2d, rowids, maps):
    mesh = plsc.VectorSubcoreMesh(core_axis_name="c", subcore_axis_name="s",
                                  num_cores=2, num_subcores=16)
    xshape = jax.ShapeDtypeStruct((_NPAD, _C2D), jnp.float32)
    fn = pl.kernel(
        _sc_body,
        out_type=[xshape, xshape, xshape],
        mesh=mesh,
        compiler_params=pltpu.CompilerParams(needs_layout_passes=False,
                                             use_tc_tiling_on_sc=False),
        scratch_types=(
            [pltpu.VMEM((4, 128), jnp.int32)] * 2
            + [pltpu.VMEM((_K, 16), jnp.int32)] * 2
            + [pltpu.VMEM((2, 128), jnp.int32)] * 6
            + [pltpu.VMEM((_K, _C2D), jnp.float32)] * 6
            + [pltpu.SemaphoreType.DMA] * 8
        ),
    )
    return fn(table2d, rowids, maps)


def _scale_shift(stats_ref, g_ref, be_ref):
    inv_n = jnp.float32(1.0 / _N)
    mu = stats_ref[0:1, :] * inv_n
    var = stats_ref[1:2, :] * inv_n - mu * mu
    sc = g_ref[...] * lax.rsqrt(var + _EPS)
    sh = be_ref[...] - mu * sc
    return sc, sh


def _accum(i, y, acc_ref, stats_ref):
    @pl.when(i == 0)
    def _():
        acc_ref[...] = jnp.zeros_like(acc_ref)

    acc_ref[0:1, :] += jnp.sum(y, axis=0, keepdims=True)
    acc_ref[1:2, :] += jnp.sum(y * y, axis=0, keepdims=True)

    @pl.when(i == _NTB - 1)
    def _():
        stats_ref[...] = acc_ref[...]


def _p1_body(x0, x1, x2, w1, b1, y1_out, stats_out, acc):
    i = pl.program_id(0)
    y = (jnp.dot(x0[...], w1[0:64, :], preferred_element_type=jnp.float32)
         + jnp.dot(x1[...], w1[64:128, :], preferred_element_type=jnp.float32)
         + jnp.dot(x2[...], w1[128:192, :], preferred_element_type=jnp.float32)
         + b1[...])
    y1_out[...] = y
    _accum(i, y, acc, stats_out)


def _h1(y1_ref, s1, g1, be1):
    sc1, sh1 = _scale_shift(s1, g1, be1)
    return jnp.maximum(y1_ref[...] * sc1 + sh1, 0.0)


def _p2_body(y1, s1, g1, be1, w2, b2, stats_out, acc):
    i = pl.program_id(0)
    h = _h1(y1, s1, g1, be1)
    y2 = jnp.dot(h, w2[...], preferred_element_type=jnp.float32) + b2[...]
    _accum(i, y2, acc, stats_out)


def _y3(y1, f3d, s1, g1, be1, w2, b2, s2, g2, be2, w3, b3):
    h = _h1(y1, s1, g1, be1)
    y2 = jnp.dot(h, w2[...], preferred_element_type=jnp.float32) + b2[...]
    sc2, sh2 = _scale_shift(s2, g2, be2)
    h2 = jnp.maximum(y2 * sc2 + sh2, 0.0)
    return (jnp.dot(f3d[...], w3[0:_D3, :], preferred_element_type=jnp.float32)
            + jnp.dot(h2, w3[_D3:2 * _D3, :], preferred_element_type=jnp.float32)
            + b3[...])


def _p3_body(y1, f3d, s1, g1, be1, w2, b2, s2, g2, be2, w3, b3,
             stats_out, acc):
    i = pl.program_id(0)
    y3 = _y3(y1, f3d, s1, g1, be1, w2, b2, s2, g2, be2, w3, b3)
    _accum(i, y3, acc, stats_out)


def _p4_body(y1, f3d, s1, g1, be1, w2, b2, s2, g2, be2, w3, b3, s3, g3, be3,
             out):
    y3 = _y3(y1, f3d, s1, g1, be1, w2, b2, s2, g2, be2, w3, b3)
    sc3, sh3 = _scale_shift(s3, g3, be3)
    out[...] = jnp.maximum(y3 * sc3 + sh3, 0.0)


def _row_spec(c):
    return pl.BlockSpec((_BN, c), lambda i: (i, 0))


def _full_spec(r, c):
    return pl.BlockSpec((r, c), lambda i: (0, 0))


def _stats_spec(c):
    return pl.BlockSpec((2, c), lambda i: (0, 0))


def kernel(feat_2d_all, sparse_feat_3d_F, links, coords_map_in,
           coords_map_out, W1, b1, g1, be1, W2, b2, g2, be2, W3, b3, g3,
           be3):
    feat6 = feat_2d_all.reshape(_NVB, _C2D, _HW)
    table = _build_table(feat6).reshape(_TROWS, _C2D)

    links2d = links.reshape(_N, 4 * _V)
    pad = _NPAD - _N
    cin_p = jnp.concatenate(
        [coords_map_in, jnp.zeros((pad,), jnp.int32)]).reshape(-1, 2, 128)
    cout_p = jnp.concatenate(
        [coords_map_out,
         jnp.arange(_N, _NPAD, dtype=jnp.int32)]).reshape(-1, 2, 128)
    maps = jnp.concatenate([cin_p, cout_p], axis=1).reshape(-1, 128)

    rowids = _build_rowids(links2d)
    x0, x1, x2 = _sc_gather(table, rowids, maps)

    r = lambda a: a.reshape(1, -1)
    b1r, g1r, be1r = r(b1), r(g1), r(be1)
    b2r, g2r, be2r = r(b2), r(g2), r(be2)
    b3r, g3r, be3r = r(b3), r(g3), r(be3)

    y1, s1 = pl.pallas_call(
        _p1_body,
        grid=(_NTB,),
        in_specs=[_row_spec(_C2D), _row_spec(_C2D), _row_spec(_C2D),
                  _full_spec(_V * _C2D, _C2D), _full_spec(1, _C2D)],
        out_specs=[_row_spec(_C2D), _stats_spec(_C2D)],
        out_shape=[jax.ShapeDtypeStruct((_N, _C2D), jnp.float32),
                   jax.ShapeDtypeStruct((2, _C2D), jnp.float32)],
        scratch_shapes=[pltpu.VMEM((2, _C2D), jnp.float32)],
    )(x0, x1, x2, W1, b1r)

    s2 = pl.pallas_call(
        _p2_body,
        grid=(_NTB,),
        in_specs=[_row_spec(_C2D), _stats_spec(_C2D), _full_spec(1, _C2D),
                  _full_spec(1, _C2D), _full_spec(_C2D, _D3),
                  _full_spec(1, _D3)],
        out_specs=_stats_spec(_D3),
        out_shape=jax.ShapeDtypeStruct((2, _D3), jnp.float32),
        scratch_shapes=[pltpu.VMEM((2, _D3), jnp.float32)],
    )(y1, s1, g1r, be1r, W2, b2r)

    big_in = [_row_spec(_C2D), _row_spec(_D3), _stats_spec(_C2D),
              _full_spec(1, _C2D), _full_spec(1, _C2D),
              _full_spec(_C2D, _D3), _full_spec(1, _D3), _stats_spec(_D3),
              _full_spec(1, _D3), _full_spec(1, _D3),
              _full_spec(2 * _D3, _D3), _full_spec(1, _D3)]

    s3 = pl.pallas_call(
        _p3_body,
        grid=(_NTB,),
        in_specs=big_in,
        out_specs=_stats_spec(_D3),
        out_shape=jax.ShapeDtypeStruct((2, _D3), jnp.float32),
        scratch_shapes=[pltpu.VMEM((2, _D3), jnp.float32)],
    )(y1, sparse_feat_3d_F, s1, g1r, be1r, W2, b2r, s2, g2r, be2r, W3, b3r)

    out = pl.pallas_call(
        _p4_body,
        grid=(_NTB,),
        in_specs=big_in + [_stats_spec(_D3), _full_spec(1, _D3),
                           _full_spec(1, _D3)],
        out_specs=_row_spec(_D3),
        out_shape=jax.ShapeDtypeStruct((_N, _D3), jnp.float32),
    )(y1, sparse_feat_3d_F, s1, g1r, be1r, W2, b2r, s2, g2r, be2r, W3, b3r,
      s3, g3r, be3r)

    return out


# speedup vs baseline: 1.1519x; 1.1368x over previous
import functools

import jax
import jax.numpy as jnp
from jax import lax
from jax.experimental import pallas as pl
from jax.experimental.pallas import tpu as pltpu
from jax.experimental.pallas import tpu_sc as plsc

_V = 3
_C2D = 64
_D3 = 96
_BS = 2
_H = 120
_W = 160
_N = 100000
_HW = _H * _W
_PAD_ROWS = 1920
_SLAB = _HW + _PAD_ROWS
_ZROW = _HW
_NVB = _V * _BS
_TROWS = _NVB * _SLAB

_NW = 32
_K = 256
_NBLK = 14
_NRUN = 13
_NPAD = _NW * _K * _NBLK

_BN = 2000
_NTB = _N // _BN
_EPS = 1e-5


_TBLK = 1920


def _transpose_body(in_ref, out_ref):
    j = pl.program_id(1)

    @pl.when(j < 10)
    def _():
        t = in_ref[0].T
        out_ref[0] = jnp.concatenate(
            [t[0:_TBLK // 2, :], t[_TBLK // 2:_TBLK, :]], axis=1)

    @pl.when(j == 10)
    def _():
        out_ref[0] = jnp.zeros((_TBLK // 2, 2 * _C2D), jnp.float32)


def _build_table(feat6):
    return pl.pallas_call(
        _transpose_body,
        grid=(_NVB, _SLAB // _TBLK),
        in_specs=[pl.BlockSpec((1, _C2D, _TBLK),
                               lambda i, j: (i, 0, jnp.minimum(j, 9)))],
        out_specs=pl.BlockSpec((1, _TBLK // 2, 2 * _C2D),
                               lambda i, j: (i, j, 0)),
        out_shape=jax.ShapeDtypeStruct((_NVB, _SLAB // 2, 2 * _C2D),
                                       jnp.float32),
    )(feat6)


def _sc_body(table_hbm, rowids_hbm, maps_hbm,
             x0_hbm, x1_hbm, x2_hbm,
             m0, m1, r0, r1,
             i00, i01, i02, i10, i11, i12,
             f00, f01, f02, f10, f11, f12,
             ms0, ms1, gs0, gs1, fs0, fs1, ss0, ss1):
    wid = lax.axis_index("s") * 2 + lax.axis_index("c")
    maps = (m0, m1)
    rids = (r0, r1)
    idxs = ((i00, i01, i02), (i10, i11, i12))
    feats = ((f00, f01, f02), (f10, f11, f12))
    msems = (ms0, ms1)
    gsems = (gs0, gs1)
    fsems = (fs0, fs1)
    ssems = (ss0, ss1)
    xs = (x0_hbm, x1_hbm, x2_hbm)

    def map_copy(b, p):
        gb = wid + _NW * b
        return pltpu.make_async_copy(
            maps_hbm.at[pl.ds(gb * 4, 4)], maps[p], msems[p])

    def rid_copies(p):
        return [pltpu.make_async_copy(
            rowids_hbm.at[maps[p].at[sub]],
            rids[p].at[pl.ds(sub * 128, 128)], gsems[p]) for sub in range(2)]

    def compute_idx(p):
        for sub in range(2):
            for g in range(8):
                rows = lax.iota(jnp.int32, 16) + (sub * 128 + g * 16)
                for v in range(_V):
                    cv = jnp.full((16,), v, jnp.int32)
                    idxs[p][v][sub, pl.ds(g * 16, 16)] = plsc.load_gather(
                        rids[p], [rows, cv])

    def fgather_copies(p):
        return [pltpu.make_async_copy(
            table_hbm.at[idxs[p][v].at[sub]],
            feats[p][v].at[pl.ds(sub * 128, 128)], fsems[p])
            for v in range(_V) for sub in range(2)]

    def scatter_copies(p):
        return [pltpu.make_async_copy(
            feats[p][v].at[pl.ds(sub * 128, 128)],
            xs[v].at[maps[p].at[2 + sub]], ssems[p])
            for v in range(_V) for sub in range(2)]

    def start(copies):
        for c in copies:
            c.start()

    def drain(copies):
        for c in copies:
            c.wait()

    def prefetch(b1, q):
        map_copy(b1, q).start()
        map_copy(b1, q).wait()
        start(rid_copies(q))
        drain(rid_copies(q))
        compute_idx(q)

    def body(b, p, first, prefetch_next):
        q = 1 - p
        start(fgather_copies(p))
        if not first:
            drain(scatter_copies(q))
        drain(fgather_copies(p))
        start(scatter_copies(p))
        if prefetch_next:
            prefetch(b + 1, q)

    prefetch(0, 0)
    body(0, 0, True, True)

    def chunk(c, carry):
        b = 1 + 2 * c
        body(b, 1, False, True)
        body(b + 1, 0, False, True)
        return carry

    lax.fori_loop(0, (_NRUN - 1) // 2, chunk, 0)

    drain(scatter_copies(0))


_RBN = 4096


def _rowid_body(lk_ref, out_ref):
    lk = lk_ref[...]
    cols = []
    for v in range(_V):
        b = lk[:, v:v + 1]
        vi = lk[:, 3 + v:4 + v]
        ui = lk[:, 6 + v:7 + v]
        val = lk[:, 9 + v:10 + v]
        pix = jnp.where(val > 0, vi * _W + ui, _ZROW)
        j = pix // _TBLK
        p = pix - j * _TBLK
        hi = (p >= _TBLK // 2).astype(jnp.int32)
        row = 2 * ((_TBLK // 2) * j + p - hi * (_TBLK // 2)) + hi
        cols.append((v * _BS + b) * _SLAB + row)
    cols.append(jnp.zeros((_RBN, 16 - _V), jnp.int32))
    out_ref[...] = jnp.concatenate(cols, axis=1)


def _build_rowids(links2d):
    return pl.pallas_call(
        _rowid_body,
        grid=((_N + _RBN - 1) // _RBN,),
        in_specs=[pl.BlockSpec((_RBN, 4 * _V), lambda i: (i, 0))],
        out_specs=pl.BlockSpec((_RBN, 16), lambda i: (i, 0)),
        out_shape=jax.ShapeDtypeStruct((_N, 16), jnp.int32),
    )(links2d)


def _sc_gather(table2d, rowids, maps):
    mesh = plsc.VectorSubcoreMesh(core_axis_name="c", subcore_axis_name="s",
                                  num_cores=2, num_subcores=16)
    xshape = jax.ShapeDtypeStruct((_NPAD, _C2D), jnp.float32)
    fn = pl.kernel(
        _sc_body,
        out_type=[xshape, xshape, xshape],
        mesh=mesh,
        compiler_params=pltpu.CompilerParams(needs_layout_passes=False,
                                             use_tc_tiling_on_sc=False),
        scratch_types=(
            [pltpu.VMEM((4, 128), jnp.int32)] * 2
            + [pltpu.VMEM((_K, 16), jnp.int32)] * 2
            + [pltpu.VMEM((2, 128), jnp.int32)] * 6
            + [pltpu.VMEM((_K, _C2D), jnp.float32)] * 6
            + [pltpu.SemaphoreType.DMA] * 8
        ),
    )
    return fn(table2d, rowids, maps)


def _scale_shift(stats_ref, g_ref, be_ref):
    inv_n = jnp.float32(1.0 / _N)
    mu = stats_ref[0:1, :] * inv_n
    var = stats_ref[1:2, :] * inv_n - mu * mu
    sc = g_ref[...] * lax.rsqrt(var + _EPS)
    sh = be_ref[...] - mu * sc
    return sc, sh


def _accum(i, y, acc_ref, stats_ref):
    @pl.when(i == 0)
    def _():
        acc_ref[...] = jnp.zeros_like(acc_ref)

    acc_ref[0:1, :] += jnp.sum(y, axis=0, keepdims=True)
    acc_ref[1:2, :] += jnp.sum(y * y, axis=0, keepdims=True)

    @pl.when(i == _NTB - 1)
    def _():
        stats_ref[...] = acc_ref[...]


def _accum2(i, ya, yb, acc_ref, stats_ref):
    @pl.when(i == 0)
    def _():
        acc_ref[...] = jnp.zeros_like(acc_ref)

    acc_ref[0:1, :] += jnp.sum(ya, axis=0, keepdims=True) + jnp.sum(
        yb, axis=0, keepdims=True)
    acc_ref[1:2, :] += jnp.sum(ya * ya, axis=0, keepdims=True) + jnp.sum(
        yb * yb, axis=0, keepdims=True)

    @pl.when(i == _NTB - 1)
    def _():
        stats_ref[...] = acc_ref[...]


def _dot(a, b):
    return jnp.dot(a, b, preferred_element_type=jnp.float32)


def _p1_body(x0, x1, x2, w1, b1, y1_out, stats_out, acc):
    i = pl.program_id(0)
    ys = []
    for h in range(2):
        sl = slice(h * _C2D, (h + 1) * _C2D)
        ys.append(_dot(x0[:, sl], w1[0:64, :])
                  + _dot(x1[:, sl], w1[64:128, :])
                  + _dot(x2[:, sl], w1[128:192, :]) + b1[...])
    y1_out[...] = jnp.concatenate(ys, axis=1)
    _accum2(i, ys[0], ys[1], acc, stats_out)


def _y2_half(y1_ref, h, sc1, sh1, w2, b2):
    hh = jnp.maximum(y1_ref[:, h * _C2D:(h + 1) * _C2D] * sc1 + sh1, 0.0)
    return _dot(hh, w2[...]) + b2[...]


def _p2_body(y1, s1, g1, be1, w2, b2, stats_out, acc):
    i = pl.program_id(0)
    sc1, sh1 = _scale_shift(s1, g1, be1)
    y2a = _y2_half(y1, 0, sc1, sh1, w2, b2)
    y2b = _y2_half(y1, 1, sc1, sh1, w2, b2)
    _accum2(i, y2a, y2b, acc, stats_out)


def _y3(y1, f3d, s1, g1, be1, w2, b2, s2, g2, be2, w3, b3):
    sc1, sh1 = _scale_shift(s1, g1, be1)
    sc2, sh2 = _scale_shift(s2, g2, be2)
    h2s = []
    for h in range(2):
        y2 = _y2_half(y1, h, sc1, sh1, w2, b2)
        h2s.append(jnp.maximum(y2 * sc2 + sh2, 0.0))
    h2 = jnp.stack(h2s, axis=1).reshape(_BN, _D3)
    return (_dot(f3d[...], w3[0:_D3, :])
            + _dot(h2, w3[_D3:2 * _D3, :]) + b3[...])


def _p3_body(y1, f3d, s1, g1, be1, w2, b2, s2, g2, be2, w3, b3,
             stats_out, acc):
    i = pl.program_id(0)
    y3 = _y3(y1, f3d, s1, g1, be1, w2, b2, s2, g2, be2, w3, b3)
    _accum(i, y3, acc, stats_out)


def _p4_body(y1, f3d, s1, g1, be1, w2, b2, s2, g2, be2, w3, b3, s3, g3, be3,
             out):
    y3 = _y3(y1, f3d, s1, g1, be1, w2, b2, s2, g2, be2, w3, b3)
    sc3, sh3 = _scale_shift(s3, g3, be3)
    out[...] = jnp.maximum(y3 * sc3 + sh3, 0.0)


def _row_spec(c):
    return pl.BlockSpec((_BN, c), lambda i: (i, 0))


def _full_spec(r, c):
    return pl.BlockSpec((r, c), lambda i: (0, 0))


def _stats_spec(c):
    return pl.BlockSpec((2, c), lambda i: (0, 0))


def _half_spec():
    return pl.BlockSpec((_BN // 2, 128), lambda i: (i, 0))


def kernel(feat_2d_all, sparse_feat_3d_F, links, coords_map_in,
           coords_map_out, W1, b1, g1, be1, W2, b2, g2, be2, W3, b3, g3,
           be3):
    feat6 = feat_2d_all.reshape(_NVB, _C2D, _HW)
    table = _build_table(feat6).reshape(_TROWS, _C2D)

    links2d = links.reshape(_N, 4 * _V)
    pad = _NPAD - _N
    cin_p = jnp.concatenate(
        [coords_map_in, jnp.zeros((pad,), jnp.int32)]).reshape(-1, 2, 128)
    cout_p = jnp.concatenate(
        [coords_map_out,
         jnp.arange(_N, _NPAD, dtype=jnp.int32)]).reshape(-1, 2, 128)
    maps = jnp.concatenate([cin_p, cout_p], axis=1).reshape(-1, 128)

    rowids = _build_rowids(links2d).reshape(_N, 16)
    x0, x1, x2 = _sc_gather(table, rowids, maps)
    x0r = x0.reshape(_NPAD // 2, 128)
    x1r = x1.reshape(_NPAD // 2, 128)
    x2r = x2.reshape(_NPAD // 2, 128)

    r = lambda a: a.reshape(1, -1)
    b1r, g1r, be1r = r(b1), r(g1), r(be1)
    b2r, g2r, be2r = r(b2), r(g2), r(be2)
    b3r, g3r, be3r = r(b3), r(g3), r(be3)

    y1, s1 = pl.pallas_call(
        _p1_body,
        grid=(_NTB,),
        in_specs=[_half_spec(), _half_spec(), _half_spec(),
                  _full_spec(_V * _C2D, _C2D), _full_spec(1, _C2D)],
        out_specs=[_half_spec(), _stats_spec(_C2D)],
        out_shape=[jax.ShapeDtypeStruct((_N // 2, 128), jnp.float32),
                   jax.ShapeDtypeStruct((2, _C2D), jnp.float32)],
        scratch_shapes=[pltpu.VMEM((2, _C2D), jnp.float32)],
    )(x0r, x1r, x2r, W1, b1r)

    s2 = pl.pallas_call(
        _p2_body,
        grid=(_NTB,),
        in_specs=[_half_spec(), _stats_spec(_C2D), _full_spec(1, _C2D),
                  _full_spec(1, _C2D), _full_spec(_C2D, _D3),
                  _full_spec(1, _D3)],
        out_specs=_stats_spec(_D3),
        out_shape=jax.ShapeDtypeStruct((2, _D3), jnp.float32),
        scratch_shapes=[pltpu.VMEM((2, _D3), jnp.float32)],
    )(y1, s1, g1r, be1r, W2, b2r)

    big_in = [_half_spec(), _row_spec(_D3), _stats_spec(_C2D),
              _full_spec(1, _C2D), _full_spec(1, _C2D),
              _full_spec(_C2D, _D3), _full_spec(1, _D3), _stats_spec(_D3),
              _full_spec(1, _D3), _full_spec(1, _D3),
              _full_spec(2 * _D3, _D3), _full_spec(1, _D3)]

    s3 = pl.pallas_call(
        _p3_body,
        grid=(_NTB,),
        in_specs=big_in,
        out_specs=_stats_spec(_D3),
        out_shape=jax.ShapeDtypeStruct((2, _D3), jnp.float32),
        scratch_shapes=[pltpu.VMEM((2, _D3), jnp.float32)],
    )(y1, sparse_feat_3d_F, s1, g1r, be1r, W2, b2r, s2, g2r, be2r, W3, b3r)

    out = pl.pallas_call(
        _p4_body,
        grid=(_NTB,),
        in_specs=big_in + [_stats_spec(_D3), _full_spec(1, _D3),
                           _full_spec(1, _D3)],
        out_specs=_row_spec(_D3),
        out_shape=jax.ShapeDtypeStruct((_N, _D3), jnp.float32),
    )(y1, sparse_feat_3d_F, s1, g1r, be1r, W2, b2r, s2, g2r, be2r, W3, b3r,
      s3, g3r, be3r)

    return out


# speedup vs baseline: 1.1957x; 1.0380x over previous
import functools

import jax
import jax.numpy as jnp
from jax import lax
from jax.experimental import pallas as pl
from jax.experimental.pallas import tpu as pltpu
from jax.experimental.pallas import tpu_sc as plsc

_V = 3
_C2D = 64
_D3 = 96
_BS = 2
_H = 120
_W = 160
_N = 100000
_HW = _H * _W
_PAD_ROWS = 1920
_SLAB = _HW + _PAD_ROWS
_ZROW = _HW
_NVB = _V * _BS
_TROWS = _NVB * _SLAB

_NW = 32
_K = 256
_NBLK = 14
_NRUN = 13
_NPAD = _NW * _K * _NBLK

_BN = 4000
_NTB = _N // _BN
_EPS = 1e-5


_TBLK = 1920


def _transpose_body(in_ref, out_ref):
    j = pl.program_id(1)

    @pl.when(j < 10)
    def _():
        t = in_ref[0].T
        out_ref[0] = jnp.concatenate(
            [t[0:_TBLK // 2, :], t[_TBLK // 2:_TBLK, :]], axis=1)

    @pl.when(j == 10)
    def _():
        out_ref[0] = jnp.zeros((_TBLK // 2, 2 * _C2D), jnp.float32)


def _build_table(feat6):
    return pl.pallas_call(
        _transpose_body,
        grid=(_NVB, _SLAB // _TBLK),
        in_specs=[pl.BlockSpec((1, _C2D, _TBLK),
                               lambda i, j: (i, 0, jnp.minimum(j, 9)))],
        out_specs=pl.BlockSpec((1, _TBLK // 2, 2 * _C2D),
                               lambda i, j: (i, j, 0)),
        out_shape=jax.ShapeDtypeStruct((_NVB, _SLAB // 2, 2 * _C2D),
                                       jnp.float32),
    )(feat6)


def _sc_body(table_hbm, rowids_hbm, maps_hbm,
             x0_hbm, x1_hbm, x2_hbm,
             m0, m1, r0, r1,
             i00, i01, i02, i10, i11, i12,
             f00, f01, f02, f10, f11, f12,
             ms0, ms1, gs0, gs1, fs0, fs1, ss0, ss1):
    wid = lax.axis_index("s") * 2 + lax.axis_index("c")
    maps = (m0, m1)
    rids = (r0, r1)
    idxs = ((i00, i01, i02), (i10, i11, i12))
    feats = ((f00, f01, f02), (f10, f11, f12))
    msems = (ms0, ms1)
    gsems = (gs0, gs1)
    fsems = (fs0, fs1)
    ssems = (ss0, ss1)
    xs = (x0_hbm, x1_hbm, x2_hbm)

    def map_copy(b, p):
        gb = wid + _NW * b
        return pltpu.make_async_copy(
            maps_hbm.at[pl.ds(gb * 4, 4)], maps[p], msems[p])

    def rid_copies(p):
        return [pltpu.make_async_copy(
            rowids_hbm.at[maps[p].at[sub]],
            rids[p].at[pl.ds(sub * 128, 128)], gsems[p]) for sub in range(2)]

    def compute_idx(p):
        for sub in range(2):
            for g in range(8):
                rows = lax.iota(jnp.int32, 16) + (sub * 128 + g * 16)
                for v in range(_V):
                    cv = jnp.full((16,), v, jnp.int32)
                    idxs[p][v][sub, pl.ds(g * 16, 16)] = plsc.load_gather(
                        rids[p], [rows, cv])

    def fgather_copies(p):
        return [pltpu.make_async_copy(
            table_hbm.at[idxs[p][v].at[sub]],
            feats[p][v].at[pl.ds(sub * 128, 128)], fsems[p])
            for v in range(_V) for sub in range(2)]

    def scatter_copies(p):
        return [pltpu.make_async_copy(
            feats[p][v].at[pl.ds(sub * 128, 128)],
            xs[v].at[maps[p].at[2 + sub]], ssems[p])
            for v in range(_V) for sub in range(2)]

    def start(copies):
        for c in copies:
            c.start()

    def drain(copies):
        for c in copies:
            c.wait()

    def prefetch(b1, q):
        map_copy(b1, q).start()
        map_copy(b1, q).wait()
        start(rid_copies(q))
        drain(rid_copies(q))
        compute_idx(q)

    def body(b, p, first, prefetch_next):
        q = 1 - p
        start(fgather_copies(p))
        if not first:
            drain(scatter_copies(q))
        drain(fgather_copies(p))
        start(scatter_copies(p))
        if prefetch_next:
            prefetch(b + 1, q)

    prefetch(0, 0)
    body(0, 0, True, True)

    def chunk(c, carry):
        b = 1 + 2 * c
        body(b, 1, False, True)
        body(b + 1, 0, False, True)
        return carry

    lax.fori_loop(0, (_NRUN - 1) // 2, chunk, 0)

    drain(scatter_copies(0))


_RBN = 4096


def _rowid_body(lk_ref, out_ref):
    lk = lk_ref[...]
    cols = []
    for v in range(_V):
        b = lk[:, v:v + 1]
        vi = lk[:, 3 + v:4 + v]
        ui = lk[:, 6 + v:7 + v]
        val = lk[:, 9 + v:10 + v]
        pix = jnp.where(val > 0, vi * _W + ui, _ZROW)
        j = pix // _TBLK
        p = pix - j * _TBLK
        hi = (p >= _TBLK // 2).astype(jnp.int32)
        row = 2 * ((_TBLK // 2) * j + p - hi * (_TBLK // 2)) + hi
        cols.append((v * _BS + b) * _SLAB + row)
    cols.append(jnp.zeros((_RBN, 16 - _V), jnp.int32))
    out_ref[...] = jnp.concatenate(cols, axis=1)


def _build_rowids(links2d):
    return pl.pallas_call(
        _rowid_body,
        grid=((_N + _RBN - 1) // _RBN,),
        in_specs=[pl.BlockSpec((_RBN, 4 * _V), lambda i: (i, 0))],
        out_specs=pl.BlockSpec((_RBN, 16), lambda i: (i, 0)),
        out_shape=jax.ShapeDtypeStruct((_N, 16), jnp.int32),
    )(links2d)


def _sc_gather(table2d, rowids, maps):
    mesh = plsc.VectorSubcoreMesh(core_axis_name="c", subcore_axis_name="s",
                                  num_cores=2, num_subcores=16)
    xshape = jax.ShapeDtypeStruct((_NPAD, _C2D), jnp.float32)
    fn = pl.kernel(
        _sc_body,
        out_type=[xshape, xshape, xshape],
        mesh=mesh,
        compiler_params=pltpu.CompilerParams(needs_layout_passes=False,
                                             use_tc_tiling_on_sc=False),
        scratch_types=(
            [pltpu.VMEM((4, 128), jnp.int32)] * 2
            + [pltpu.VMEM((_K, 16), jnp.int32)] * 2
            + [pltpu.VMEM((2, 128), jnp.int32)] * 6
            + [pltpu.VMEM((_K, _C2D), jnp.float32)] * 6
            + [pltpu.SemaphoreType.DMA] * 8
        ),
    )
    return fn(table2d, rowids, maps)


def _scale_shift(stats_ref, g_ref, be_ref):
    inv_n = jnp.float32(1.0 / _N)
    mu = stats_ref[0:1, :] * inv_n
    var = stats_ref[1:2, :] * inv_n - mu * mu
    sc = g_ref[...] * lax.rsqrt(var + _EPS)
    sh = be_ref[...] - mu * sc
    return sc, sh


def _accum(i, y, acc_ref, stats_ref):
    @pl.when(i == 0)
    def _():
        acc_ref[...] = jnp.zeros_like(acc_ref)

    acc_ref[0:1, :] += jnp.sum(y, axis=0, keepdims=True)
    acc_ref[1:2, :] += jnp.sum(y * y, axis=0, keepdims=True)

    @pl.when(i == _NTB - 1)
    def _():
        stats_ref[...] = acc_ref[...]


def _accum2(i, ya, yb, acc_ref, stats_ref):
    @pl.when(i == 0)
    def _():
        acc_ref[...] = jnp.zeros_like(acc_ref)

    acc_ref[0:1, :] += jnp.sum(ya, axis=0, keepdims=True) + jnp.sum(
        yb, axis=0, keepdims=True)
    acc_ref[1:2, :] += jnp.sum(ya * ya, axis=0, keepdims=True) + jnp.sum(
        yb * yb, axis=0, keepdims=True)

    @pl.when(i == _NTB - 1)
    def _():
        stats_ref[...] = acc_ref[...]


def _dot(a, b):
    return jnp.dot(a, b, preferred_element_type=jnp.float32)


def _p1_body(x0, x1, x2, w1, b1, y1_out, stats_out, acc):
    i = pl.program_id(0)
    ys = []
    for h in range(2):
        sl = slice(h * _C2D, (h + 1) * _C2D)
        ys.append(_dot(x0[:, sl], w1[0:64, :])
                  + _dot(x1[:, sl], w1[64:128, :])
                  + _dot(x2[:, sl], w1[128:192, :]) + b1[...])
    y1_out[...] = jnp.concatenate(ys, axis=1)
    _accum2(i, ys[0], ys[1], acc, stats_out)


def _y2_half(y1_ref, h, sc1, sh1, w2, b2):
    hh = jnp.maximum(y1_ref[:, h * _C2D:(h + 1) * _C2D] * sc1 + sh1, 0.0)
    return _dot(hh, w2[...]) + b2[...]


def _p2_body(y1, s1, g1, be1, w2, b2, stats_out, acc):
    i = pl.program_id(0)
    sc1, sh1 = _scale_shift(s1, g1, be1)
    y2a = _y2_half(y1, 0, sc1, sh1, w2, b2)
    y2b = _y2_half(y1, 1, sc1, sh1, w2, b2)
    _accum2(i, y2a, y2b, acc, stats_out)


def _y3(y1, f3d, s1, g1, be1, w2, b2, s2, g2, be2, w3, b3):
    sc1, sh1 = _scale_shift(s1, g1, be1)
    sc2, sh2 = _scale_shift(s2, g2, be2)
    h2s = []
    for h in range(2):
        y2 = _y2_half(y1, h, sc1, sh1, w2, b2)
        h2s.append(jnp.maximum(y2 * sc2 + sh2, 0.0))
    h2 = jnp.stack(h2s, axis=1).reshape(_BN, _D3)
    return (_dot(f3d[...], w3[0:_D3, :])
            + _dot(h2, w3[_D3:2 * _D3, :]) + b3[...])


def _p3_body(y1, f3d, s1, g1, be1, w2, b2, s2, g2, be2, w3, b3,
             stats_out, acc):
    i = pl.program_id(0)
    y3 = _y3(y1, f3d, s1, g1, be1, w2, b2, s2, g2, be2, w3, b3)
    _accum(i, y3, acc, stats_out)


def _p4_body(y1, f3d, s1, g1, be1, w2, b2, s2, g2, be2, w3, b3, s3, g3, be3,
             out):
    y3 = _y3(y1, f3d, s1, g1, be1, w2, b2, s2, g2, be2, w3, b3)
    sc3, sh3 = _scale_shift(s3, g3, be3)
    out[...] = jnp.maximum(y3 * sc3 + sh3, 0.0)


def _row_spec(c):
    return pl.BlockSpec((_BN, c), lambda i: (i, 0))


def _full_spec(r, c):
    return pl.BlockSpec((r, c), lambda i: (0, 0))


def _stats_spec(c):
    return pl.BlockSpec((2, c), lambda i: (0, 0))


def _half_spec():
    return pl.BlockSpec((_BN // 2, 128), lambda i: (i, 0))


def kernel(feat_2d_all, sparse_feat_3d_F, links, coords_map_in,
           coords_map_out, W1, b1, g1, be1, W2, b2, g2, be2, W3, b3, g3,
           be3):
    feat6 = feat_2d_all.reshape(_NVB, _C2D, _HW)
    table = _build_table(feat6).reshape(_TROWS, _C2D)

    links2d = links.reshape(_N, 4 * _V)
    pad = _NPAD - _N
    cin_p = jnp.concatenate(
        [coords_map_in, jnp.zeros((pad,), jnp.int32)]).reshape(-1, 2, 128)
    cout_p = jnp.concatenate(
        [coords_map_out,
         jnp.arange(_N, _NPAD, dtype=jnp.int32)]).reshape(-1, 2, 128)
    maps = jnp.concatenate([cin_p, cout_p], axis=1).reshape(-1, 128)

    rowids = _build_rowids(links2d).reshape(_N, 16)
    x0, x1, x2 = _sc_gather(table, rowids, maps)
    x0r = x0.reshape(_NPAD // 2, 128)
    x1r = x1.reshape(_NPAD // 2, 128)
    x2r = x2.reshape(_NPAD // 2, 128)

    r = lambda a: a.reshape(1, -1)
    b1r, g1r, be1r = r(b1), r(g1), r(be1)
    b2r, g2r, be2r = r(b2), r(g2), r(be2)
    b3r, g3r, be3r = r(b3), r(g3), r(be3)

    y1, s1 = pl.pallas_call(
        _p1_body,
        grid=(_NTB,),
        in_specs=[_half_spec(), _half_spec(), _half_spec(),
                  _full_spec(_V * _C2D, _C2D), _full_spec(1, _C2D)],
        out_specs=[_half_spec(), _stats_spec(_C2D)],
        out_shape=[jax.ShapeDtypeStruct((_N // 2, 128), jnp.float32),
                   jax.ShapeDtypeStruct((2, _C2D), jnp.float32)],
        scratch_shapes=[pltpu.VMEM((2, _C2D), jnp.float32)],
    )(x0r, x1r, x2r, W1, b1r)

    s2 = pl.pallas_call(
        _p2_body,
        grid=(_NTB,),
        in_specs=[_half_spec(), _stats_spec(_C2D), _full_spec(1, _C2D),
                  _full_spec(1, _C2D), _full_spec(_C2D, _D3),
                  _full_spec(1, _D3)],
        out_specs=_stats_spec(_D3),
        out_shape=jax.ShapeDtypeStruct((2, _D3), jnp.float32),
        scratch_shapes=[pltpu.VMEM((2, _D3), jnp.float32)],
    )(y1, s1, g1r, be1r, W2, b2r)

    big_in = [_half_spec(), _row_spec(_D3), _stats_spec(_C2D),
              _full_spec(1, _C2D), _full_spec(1, _C2D),
              _full_spec(_C2D, _D3), _full_spec(1, _D3), _stats_spec(_D3),
              _full_spec(1, _D3), _full_spec(1, _D3),
              _full_spec(2 * _D3, _D3), _full_spec(1, _D3)]

    s3 = pl.pallas_call(
        _p3_body,
        grid=(_NTB,),
        in_specs=big_in,
        out_specs=_stats_spec(_D3),
        out_shape=jax.ShapeDtypeStruct((2, _D3), jnp.float32),
        scratch_shapes=[pltpu.VMEM((2, _D3), jnp.float32)],
    )(y1, sparse_feat_3d_F, s1, g1r, be1r, W2, b2r, s2, g2r, be2r, W3, b3r)

    out = pl.pallas_call(
        _p4_body,
        grid=(_NTB,),
        in_specs=big_in + [_stats_spec(_D3), _full_spec(1, _D3),
                           _full_spec(1, _D3)],
        out_specs=_row_spec(_D3),
        out_shape=jax.ShapeDtypeStruct((_N, _D3), jnp.float32),
    )(y1, sparse_feat_3d_F, s1, g1r, be1r, W2, b2r, s2, g2r, be2r, W3, b3r,
      s3, g3r, be3r)

    return out
